# baseline (speedup 1.0000x reference)
import functools

import jax
import jax.numpy as jnp
from jax import lax
from jax.experimental import pallas as pl
from jax.experimental.pallas import tpu as pltpu
from jax.experimental.pallas import tpu_sc as plsc

_N_TOKENS = 16384
_IN_DIM = 2048
_HIDDEN = 64
_OUT_DIM = 16
_BLOCK = 2048

_NUM_WORKERS = 32
_ROWS_PER_WORKER = _N_TOKENS // _NUM_WORKERS


def _mlp_body(x_ref, w1_ref, w2_ref, w3_ref, logits_ref):
    h = jnp.maximum(
        jnp.dot(x_ref[...], w1_ref[...], preferred_element_type=jnp.float32),
        0.0)
    h = jnp.maximum(
        jnp.dot(h, w2_ref[...], preferred_element_type=jnp.float32), 0.0)
    logits_ref[...] = jnp.dot(h, w3_ref[...],
                              preferred_element_type=jnp.float32)


def _logits_tc(x, We1, We2, We3):
    n_blocks = _N_TOKENS // _BLOCK
    return pl.pallas_call(
        _mlp_body,
        grid=(n_blocks,),
        in_specs=[
            pl.BlockSpec((_BLOCK, _IN_DIM), lambda i: (i, 0)),
            pl.BlockSpec((_IN_DIM, _HIDDEN), lambda i: (0, 0)),
            pl.BlockSpec((_HIDDEN, _HIDDEN), lambda i: (0, 0)),
            pl.BlockSpec((_HIDDEN, _OUT_DIM), lambda i: (0, 0)),
        ],
        out_specs=pl.BlockSpec((_BLOCK, _OUT_DIM), lambda i: (i, 0)),
        out_shape=jax.ShapeDtypeStruct((_N_TOKENS, _OUT_DIM), jnp.float32),
        compiler_params=pltpu.CompilerParams(
            dimension_semantics=("arbitrary",),
        ),
    )(x, We1, We2, We3)


def _route_body(logits_hbm, gates_hbm, loadp_hbm, idxp_hbm,
                lrows, grows, irows, lstage, loads_v, sem):
    c = lax.axis_index("c")
    s = lax.axis_index("s")
    wid = s * 2 + c
    base = wid * _ROWS_PER_WORKER

    pltpu.async_copy(logits_hbm.at[pl.ds(base, _ROWS_PER_WORKER)], lrows,
                     sem).wait()

    lane = lax.iota(jnp.int32, _OUT_DIM)
    zid = jnp.zeros((_OUT_DIM,), jnp.int32)
    oid = jnp.full((_OUT_DIM,), 1, jnp.int32)
    ones = jnp.full((_OUT_DIM,), 1, jnp.int32)
    top2 = lane < 2

    loads_v[0] = zid

    def row_body(r, carry):
        row = lrows[r]
        v_sorted, id_sorted = plsc.sort_key_val(row, lane, descending=True)
        v1 = v_sorted.at[zid].get(mode="promise_in_bounds")
        e = jnp.exp(v_sorted - v1)
        e2 = e.at[oid].get(mode="promise_in_bounds")
        gvals = jnp.where(top2, e / (1.0 + e2), 0.0)
        plsc.store_scatter(grows, [jnp.full((_OUT_DIM,), r), id_sorted],
                           gvals)
        irows[r] = jnp.where(top2, id_sorted, 0)
        plsc.addupdate_scatter(loads_v, [zid, id_sorted], ones,
                               mask=top2 & (gvals > 0.0))
        return carry

    lax.fori_loop(0, _ROWS_PER_WORKER, row_body, 0, unroll=8)
    lstage[0] = loads_v[0]

    pltpu.async_copy(grows, gates_hbm.at[pl.ds(base, _ROWS_PER_WORKER)],
                     sem).wait()
    pltpu.async_copy(irows, idxp_hbm.at[pl.ds(base, _ROWS_PER_WORKER)],
                     sem).wait()
    pltpu.async_copy(lstage, loadp_hbm.at[pl.ds(wid, 1)], sem).wait()


def _route_sc(logits):
    mesh = plsc.VectorSubcoreMesh(core_axis_name="c", subcore_axis_name="s")
    route = functools.partial(
        pl.kernel,
        mesh=mesh,
        out_type=[
            jax.ShapeDtypeStruct((_N_TOKENS, _OUT_DIM), jnp.float32),
            jax.ShapeDtypeStruct((_NUM_WORKERS, _OUT_DIM), jnp.int32),
            jax.ShapeDtypeStruct((_N_TOKENS, _OUT_DIM), jnp.int32),
        ],
        scratch_types=[
            pltpu.VMEM((_ROWS_PER_WORKER, _OUT_DIM), jnp.float32),
            pltpu.VMEM((_ROWS_PER_WORKER, _OUT_DIM), jnp.float32),
            pltpu.VMEM((_ROWS_PER_WORKER, _OUT_DIM), jnp.int32),
            pltpu.VMEM((1, _OUT_DIM), jnp.int32),
            pltpu.VMEM((1, _OUT_DIM), jnp.int32),
            pltpu.SemaphoreType.DMA,
        ],
        compiler_params=pltpu.CompilerParams(
            needs_layout_passes=False, use_tc_tiling_on_sc=False),
    )(_route_body)
    return route(logits)


@jax.jit
def kernel(x, We1, We2, We3, Wn1, Wn2, Wn3):
    del Wn1, Wn2, Wn3
    logits = _logits_tc(x, We1, We2, We3)
    gates, load_partials, idx_padded = _route_sc(logits)
    return (gates,
            jnp.sum(load_partials, axis=0),
            idx_padded[:, :2])
